# baseline (speedup 1.0000x reference)
import functools

import jax
import jax.numpy as jnp
from jax import lax
from jax.experimental import pallas as pl
from jax.experimental.pallas import tpu as pltpu
from jax.experimental.pallas import tpu_sc as plsc

N = 10000
E = 320000
D_IN = 128
D_EDGE = 16
HID = 16
HEADS = 5
N_CLASSES = 2

SHIFT = 8.0

NC = 2
NS = 16
LANES = 16
NW = NC * NS

EPT = E // NW
B1 = 40
B2 = 80
GSLOTS = 3
OSLOTS = 2

f32 = jnp.float32


def _tc1a_body(x_ref, wq_ref, bq_ref, wk_ref, bk_ref, wv_ref, bv_ref,
               wsk_ref, bsk_ref, qs_ref, kv_ref, skip_ref):
    x = x_ref[...]
    q = (jnp.dot(x, wq_ref[...], preferred_element_type=f32) + bq_ref[...]) * 0.25
    k = jnp.dot(x, wk_ref[...], preferred_element_type=f32) + bk_ref[...]
    v = jnp.dot(x, wv_ref[...], preferred_element_type=f32) + bv_ref[...]
    qs_ref[...] = q
    kv_ref[...] = jnp.concatenate([k, v], axis=1)
    skip_ref[...] = jnp.dot(x, wsk_ref[...], preferred_element_type=f32) + bsk_ref[...]


def _tc1a(x, Wq, bq, Wk, bk, Wv, bv, Wsk, bsk):
    R = 2000
    grid = (N // R,)
    full = lambda a: pl.BlockSpec(a.shape, lambda i: (0,) * a.ndim)
    return pl.pallas_call(
        _tc1a_body,
        grid=grid,
        in_specs=[pl.BlockSpec((R, D_IN), lambda i: (i, 0)),
                  full(Wq), full(bq), full(Wk), full(bk), full(Wv), full(bv),
                  full(Wsk), full(bsk)],
        out_specs=[pl.BlockSpec((R, 80), lambda i: (i, 0)),
                   pl.BlockSpec((R, 160), lambda i: (i, 0)),
                   pl.BlockSpec((R, HID), lambda i: (i, 0))],
        out_shape=[jax.ShapeDtypeStruct((N, 80), f32),
                   jax.ShapeDtypeStruct((N, 160), f32),
                   jax.ShapeDtypeStruct((N, HID), f32)],
    )(x, Wq, bq, Wk, bk, Wv, bv, Wsk, bsk)


def _pack_bf16(lo, hi):
    plo = jax.lax.bitcast_convert_type(lo, jnp.int32) + 0x8000
    phi = jax.lax.bitcast_convert_type(hi, jnp.int32) + 0x8000
    return jax.lax.shift_right_logical(plo, 16) | (phi & jnp.int32(-65536))


def _tc1b_body(eat_ref, wlo_ref, whi_ref, e_ref):
    eat = eat_ref[...]
    lo = jax.lax.dot_general(eat, wlo_ref[...], (((0,), (0,)), ((), ())),
                             preferred_element_type=f32)
    hi = jax.lax.dot_general(eat, whi_ref[...], (((0,), (0,)), ((), ())),
                             preferred_element_type=f32)
    e_ref[...] = _pack_bf16(lo, hi)


def _tc1b(eaT, wlo, whi):
    w = wlo.shape[1]
    R = 16000
    grid = (E // R,)
    full = lambda a: pl.BlockSpec(a.shape, lambda i: (0,) * a.ndim)
    return pl.pallas_call(
        _tc1b_body,
        grid=grid,
        in_specs=[pl.BlockSpec((D_EDGE, R), lambda i: (0, i)),
                  full(wlo), full(whi)],
        out_specs=pl.BlockSpec((R, w), lambda i: (i, 0)),
        out_shape=jax.ShapeDtypeStruct((E, w), jnp.int32),
    )(eaT, wlo, whi)


def _sc_edge_pass(edge_index, qtab, kvtab, etab, wq, wacc, packed, B):
    nj = wq // LANES
    we = etab.shape[1]
    nblk = EPT // B
    nzc = N // B
    mesh = plsc.VectorSubcoreMesh(core_axis_name="c", subcore_axis_name="s",
                                  num_cores=NC, num_subcores=NS)

    @functools.partial(
        pl.kernel,
        out_type=jax.ShapeDtypeStruct((NC, N, wacc), f32),
        mesh=mesh,
        scratch_types=[
            pltpu.VMEM((nblk, B), jnp.int32),
            pltpu.VMEM((nblk, B), jnp.int32),
            pltpu.VMEM((GSLOTS, B, wq), f32),
            pltpu.VMEM((GSLOTS, B, 2 * wq), f32),
            pltpu.VMEM((GSLOTS, B, we), jnp.int32),
            pltpu.VMEM((OSLOTS, B, wacc), f32),
            pltpu.VMEM_SHARED((N, wacc), f32),
            pltpu.SemaphoreType.DMA((GSLOTS,)),
            pltpu.SemaphoreType.DMA((OSLOTS,)),
            pltpu.SemaphoreType.DMA,
        ],
        compiler_params=pltpu.CompilerParams(needs_layout_passes=False,
                                             use_tc_tiling_on_sc=False),
    )
    def body(ei_hbm, q_hbm, kv_hbm, e_hbm, out_hbm,
             srcv, dstv, qr, kvr, er, outr, acc, gsem, ssem, isem):
        cid = lax.axis_index("c")
        sid = lax.axis_index("s")
        wid = sid * NC + cid
        iota = lax.iota(jnp.int32, LANES)
        zeros = jnp.zeros((LANES,), f32)
        tile_base = wid * EPT

        ci = pltpu.async_copy(ei_hbm.at[0, pl.ds(wid * nblk, nblk)], srcv, isem)
        cj = pltpu.async_copy(ei_hbm.at[1, pl.ds(wid * nblk, nblk)], dstv, isem)

        @pl.loop(0, B)
        def _zrow(r):
            for c in range(wacc // LANES):
                outr[0, r, pl.ds(LANES * c, LANES)] = zeros

        for t in range(-(-nzc // NS)):
            ck = sid + NS * t

            @pl.when(ck < nzc)
            def _():
                pltpu.sync_copy(outr.at[0], acc.at[pl.ds(ck * B, B)])

        ci.wait()
        cj.wait()
        plsc.subcore_barrier()

        def _issue(b, s):
            pltpu.async_copy(q_hbm.at[dstv.at[b]], qr.at[s], gsem.at[s])
            pltpu.async_copy(kv_hbm.at[srcv.at[b]], kvr.at[s], gsem.at[s])
            pltpu.async_copy(e_hbm.at[pl.ds(tile_base + b * B, B)],
                             er.at[s], gsem.at[s])

        for p in range(GSLOTS):
            _issue(p, p)

        @pl.loop(0, nblk)
        def _blk(b):
            s = lax.rem(b, GSLOTS)
            so = lax.rem(b, OSLOTS)

            pltpu.make_async_copy(q_hbm.at[dstv.at[b]], qr.at[s],
                                  gsem.at[s]).wait()
            pltpu.make_async_copy(kv_hbm.at[srcv.at[b]], kvr.at[s],
                                  gsem.at[s]).wait()
            pltpu.make_async_copy(e_hbm.at[pl.ds(tile_base + b * B, B)],
                                  er.at[s], gsem.at[s]).wait()

            @pl.when(b >= OSLOTS)
            def _():
                pltpu.make_async_copy(outr.at[so], acc.at[dstv.at[b]],
                                      ssem.at[so]).wait()

            @plsc.parallel_loop(0, B, unroll=2)
            def _edge(i):
                den = zeros
                evs = []
                for jp in range(wq // 32 + (1 if wq % 32 else 0)):
                    x = er[s, i, pl.ds(LANES * jp, LANES)]
                    evs.append(plsc.bitcast(jax.lax.shift_left(x, 16), f32))
                    if 32 * jp + 16 < wq:
                        evs.append(plsc.bitcast(x & jnp.int32(-65536), f32))
                for jh in range(nj):
                    sl = pl.ds(LANES * jh, LANES)
                    ev = evs[jh]
                    kvec = kvr[s, i, sl] + ev
                    vvec = kvr[s, i, pl.ds(wq + LANES * jh, LANES)] + ev
                    p = qr[s, i, sl] * kvec
                    pre = plsc.cumsum(p)
                    if not packed:
                        a = jnp.full((LANES,), pre[LANES - 1], f32)
                        ex = jnp.exp(a - SHIFT)
                        outr[so, i, sl] = ex * vvec
                        den = jnp.where(iota == jh, ex, den)
                    else:
                        alo = jnp.full((LANES,), pre[7], f32)
                        ahi = jnp.full((LANES,), pre[LANES - 1], f32) - alo
                        av = jnp.where(iota < 8, alo, ahi)
                        ex = jnp.exp(av - SHIFT)
                        outr[so, i, sl] = ex * vvec
                        exlo = jnp.full((LANES,), ex[0], f32)
                        den = jnp.where(iota == 2 * jh, exlo, den)
                        if 2 * jh + 1 < HEADS:
                            exhi = jnp.full((LANES,), ex[8], f32)
                            den = jnp.where(iota == 2 * jh + 1, exhi, den)
                outr[so, i, pl.ds(wq, LANES)] = den

            pltpu.make_async_copy(outr.at[so], acc.at[dstv.at[b]],
                                  ssem.at[so]).start(add=True)

            @pl.when(b + GSLOTS < nblk)
            def _():
                _issue(b + GSLOTS, s)

        for so in range(OSLOTS):
            pltpu.make_async_copy(outr.at[so], acc.at[dstv.at[0]],
                                  ssem.at[so]).wait()

        plsc.subcore_barrier()

        for t in range(-(-nzc // NS)):
            ck = sid + NS * t

            @pl.when(ck < nzc)
            def _():
                pltpu.sync_copy(acc.at[pl.ds(ck * B, B)],
                                out_hbm.at[cid, pl.ds(ck * B, B)])

    return body(edge_index.reshape(2, E // B, B), qtab, kvtab, etab)


def _tc2_body(acc_ref, skip_ref, wq_ref, bq_ref, wk_ref, bk_ref, wv_ref,
              bv_ref, wsk_ref, bsk_ref, qs_ref, kv_ref, skip2_ref):
    a = acc_ref[0] + acc_ref[1]
    r = a.shape[0]
    num = a[:, :80].reshape(r, HEADS, 16)
    den = a[:, 80:80 + HEADS]
    agg = num / (den[:, :, None] + 1e-30)
    h1 = jnp.maximum(jnp.mean(agg, axis=1) + skip_ref[...], 0.0)
    zpad = jnp.zeros((r, 8), f32)
    q = (jnp.dot(h1, wq_ref[...], preferred_element_type=f32) + bq_ref[...])
    qs_ref[...] = jnp.concatenate([q * (1.0 / jnp.sqrt(8.0)), zpad], axis=1)
    k = jnp.dot(h1, wk_ref[...], preferred_element_type=f32) + bk_ref[...]
    v = jnp.dot(h1, wv_ref[...], preferred_element_type=f32) + bv_ref[...]
    kv_ref[...] = jnp.concatenate([k, zpad, v, zpad], axis=1)
    skip2_ref[...] = jnp.dot(h1, wsk_ref[...], preferred_element_type=f32) + bsk_ref[...]


def _tc2(acc1, skip1, Wq, bq, Wk, bk, Wv, bv, Wsk, bsk):
    R = 5000
    grid = (N // R,)
    full = lambda a: pl.BlockSpec(a.shape, lambda i: (0,) * a.ndim)
    return pl.pallas_call(
        _tc2_body,
        grid=grid,
        in_specs=[pl.BlockSpec((NC, R, 96), lambda i: (0, i, 0)),
                  pl.BlockSpec((R, HID), lambda i: (i, 0)),
                  full(Wq), full(bq), full(Wk), full(bk), full(Wv), full(bv),
                  full(Wsk), full(bsk)],
        out_specs=[pl.BlockSpec((R, 48), lambda i: (i, 0)),
                   pl.BlockSpec((R, 96), lambda i: (i, 0)),
                   pl.BlockSpec((R, 8), lambda i: (i, 0))],
        out_shape=[jax.ShapeDtypeStruct((N, 48), f32),
                   jax.ShapeDtypeStruct((N, 96), f32),
                   jax.ShapeDtypeStruct((N, 8), f32)],
    )(acc1, skip1, Wq, bq, Wk, bk, Wv, bv, Wsk, bsk)


def _tc3_body(acc_ref, skip_ref, w3_ref, b3_ref, w4_ref, b4_ref, out_ref):
    a = acc_ref[0] + acc_ref[1]
    r = a.shape[0]
    num = a[:, :48].reshape(r, 6, 8)[:, :HEADS, :]
    den = a[:, 48:48 + HEADS]
    agg = num / (den[:, :, None] + 1e-30)
    h2 = jnp.maximum(jnp.mean(agg, axis=1) + skip_ref[...], 0.0)
    h3 = jnp.maximum(jnp.dot(h2, w3_ref[...], preferred_element_type=f32) + b3_ref[...], 0.0)
    out_ref[...] = jnp.dot(h3, w4_ref[...], preferred_element_type=f32) + b4_ref[...]


def _tc3(acc2, skip2, W3, b3, W4, b4):
    R = 5000
    grid = (N // R,)
    full = lambda a: pl.BlockSpec(a.shape, lambda i: (0,) * a.ndim)
    return pl.pallas_call(
        _tc3_body,
        grid=grid,
        in_specs=[pl.BlockSpec((NC, R, 64), lambda i: (0, i, 0)),
                  pl.BlockSpec((R, 8), lambda i: (i, 0)),
                  full(W3), full(b3), full(W4), full(b4)],
        out_specs=pl.BlockSpec((R, N_CLASSES), lambda i: (i, 0)),
        out_shape=jax.ShapeDtypeStruct((N, N_CLASSES), f32),
    )(acc2, skip2, W3, b3, W4, b4)


def kernel(x, edge_index, edge_attr,
           Wq1, bq1, Wk1, bk1, Wv1, bv1, We1, Wskip1, bskip1,
           Wq2, bq2, Wk2, bk2, Wv2, bv2, We2, Wskip2, bskip2,
           W3, b3, W4, b4):
    qs1, kv1, skip1 = _tc1a(x, Wq1, bq1, Wk1, bk1, Wv1, bv1, Wskip1, bskip1)
    eaT = edge_attr.T
    z16 = jnp.zeros((D_EDGE, 16), f32)
    w1lo = jnp.concatenate([We1[:, 0:16], We1[:, 32:48], We1[:, 64:80]], axis=1)
    w1hi = jnp.concatenate([We1[:, 16:32], We1[:, 48:64], z16], axis=1)
    w2lo = jnp.concatenate([We2[:, 0:16], We2[:, 32:40], z16[:, :8]], axis=1)
    w2hi = jnp.concatenate([We2[:, 16:32], z16], axis=1)
    e1 = _tc1b(eaT, w1lo, w1hi)
    e2 = _tc1b(eaT, w2lo, w2hi)

    acc1 = _sc_edge_pass(edge_index, qs1, kv1, e1, 80, 96, packed=False, B=B1)
    qs2, kv2, skip2 = _tc2(acc1, skip1, Wq2, bq2, Wk2, bk2, Wv2, bv2,
                           Wskip2, bskip2)
    acc2 = _sc_edge_pass(edge_index, qs2, kv2, e2, 48, 64, packed=True, B=B2)
    return _tc3(acc2, skip2, W3, b3, W4, b4)


# speedup vs baseline: 1.0085x; 1.0085x over previous
import functools

import jax
import jax.numpy as jnp
from jax import lax
from jax.experimental import pallas as pl
from jax.experimental.pallas import tpu as pltpu
from jax.experimental.pallas import tpu_sc as plsc

N = 10000
E = 320000
D_IN = 128
D_EDGE = 16
HID = 16
HEADS = 5
N_CLASSES = 2

SHIFT = 8.0

NC = 2
NS = 16
LANES = 16
NW = NC * NS

EPT = E // NW
B1 = 40
B2 = 80
GSLOTS = 3
OSLOTS = 2

f32 = jnp.float32


def _tc1a_body(x_ref, wq_ref, bq_ref, wk_ref, bk_ref, wv_ref, bv_ref,
               wsk_ref, bsk_ref, qs_ref, kv_ref, skip_ref):
    x = x_ref[...]
    q = (jnp.dot(x, wq_ref[...], preferred_element_type=f32) + bq_ref[...]) * 0.25
    k = jnp.dot(x, wk_ref[...], preferred_element_type=f32) + bk_ref[...]
    v = jnp.dot(x, wv_ref[...], preferred_element_type=f32) + bv_ref[...]
    qs_ref[...] = q
    kv_ref[...] = jnp.concatenate([k, v], axis=1)
    skip_ref[...] = jnp.dot(x, wsk_ref[...], preferred_element_type=f32) + bsk_ref[...]


def _tc1a(x, Wq, bq, Wk, bk, Wv, bv, Wsk, bsk):
    R = 2000
    grid = (N // R,)
    full = lambda a: pl.BlockSpec(a.shape, lambda i: (0,) * a.ndim)
    return pl.pallas_call(
        _tc1a_body,
        grid=grid,
        in_specs=[pl.BlockSpec((R, D_IN), lambda i: (i, 0)),
                  full(Wq), full(bq), full(Wk), full(bk), full(Wv), full(bv),
                  full(Wsk), full(bsk)],
        out_specs=[pl.BlockSpec((R, 80), lambda i: (i, 0)),
                   pl.BlockSpec((R, 160), lambda i: (i, 0)),
                   pl.BlockSpec((R, HID), lambda i: (i, 0))],
        out_shape=[jax.ShapeDtypeStruct((N, 80), f32),
                   jax.ShapeDtypeStruct((N, 160), f32),
                   jax.ShapeDtypeStruct((N, HID), f32)],
    )(x, Wq, bq, Wk, bk, Wv, bv, Wsk, bsk)


def _pack_bf16(lo, hi):
    plo = jax.lax.bitcast_convert_type(lo, jnp.int32) + 0x8000
    phi = jax.lax.bitcast_convert_type(hi, jnp.int32) + 0x8000
    return jax.lax.shift_right_logical(plo, 16) | (phi & jnp.int32(-65536))


def _tc1b_body(eat_ref, wlo_ref, whi_ref, e_ref):
    eat = eat_ref[...]
    lo = jax.lax.dot_general(eat, wlo_ref[...], (((0,), (0,)), ((), ())),
                             preferred_element_type=f32)
    hi = jax.lax.dot_general(eat, whi_ref[...], (((0,), (0,)), ((), ())),
                             preferred_element_type=f32)
    e_ref[...] = _pack_bf16(lo, hi)


def _tc1b(eaT, wlo, whi):
    w = wlo.shape[1]
    R = 16000
    grid = (E // R,)
    full = lambda a: pl.BlockSpec(a.shape, lambda i: (0,) * a.ndim)
    return pl.pallas_call(
        _tc1b_body,
        grid=grid,
        in_specs=[pl.BlockSpec((D_EDGE, R), lambda i: (0, i)),
                  full(wlo), full(whi)],
        out_specs=pl.BlockSpec((R, w), lambda i: (i, 0)),
        out_shape=jax.ShapeDtypeStruct((E, w), jnp.int32),
    )(eaT, wlo, whi)


def _sc_edge_pass(edge_index, qtab, kvtab, etab, wq, wacc, packed, B):
    nj = wq // LANES
    we = etab.shape[1]
    nblk = EPT // B
    nzc = N // B
    mesh = plsc.VectorSubcoreMesh(core_axis_name="c", subcore_axis_name="s",
                                  num_cores=NC, num_subcores=NS)

    @functools.partial(
        pl.kernel,
        out_type=jax.ShapeDtypeStruct((NC, N, wacc), f32),
        mesh=mesh,
        scratch_types=[
            pltpu.VMEM((nblk, B), jnp.int32),
            pltpu.VMEM((nblk, B), jnp.int32),
            pltpu.VMEM((GSLOTS, B, wq), f32),
            pltpu.VMEM((GSLOTS, B, 2 * wq), f32),
            pltpu.VMEM((GSLOTS, B, we), jnp.int32),
            pltpu.VMEM((OSLOTS, B, wacc), f32),
            pltpu.VMEM_SHARED((N, wacc), f32),
            pltpu.SemaphoreType.DMA((GSLOTS,)),
            pltpu.SemaphoreType.DMA((OSLOTS,)),
            pltpu.SemaphoreType.DMA,
        ],
        compiler_params=pltpu.CompilerParams(needs_layout_passes=False,
                                             use_tc_tiling_on_sc=False),
    )
    def body(ei_hbm, q_hbm, kv_hbm, e_hbm, out_hbm,
             srcv, dstv, qr, kvr, er, outr, acc, gsem, ssem, isem):
        cid = lax.axis_index("c")
        sid = lax.axis_index("s")
        wid = sid * NC + cid
        iota = lax.iota(jnp.int32, LANES)
        zeros = jnp.zeros((LANES,), f32)
        tile_base = wid * EPT

        ci = pltpu.async_copy(ei_hbm.at[0, pl.ds(wid * nblk, nblk)], srcv, isem)
        cj = pltpu.async_copy(ei_hbm.at[1, pl.ds(wid * nblk, nblk)], dstv, isem)

        @pl.loop(0, B)
        def _zrow(r):
            for c in range(wacc // LANES):
                outr[0, r, pl.ds(LANES * c, LANES)] = zeros

        for t in range(-(-nzc // NS)):
            ck = sid + NS * t

            @pl.when(ck < nzc)
            def _():
                pltpu.sync_copy(outr.at[0], acc.at[pl.ds(ck * B, B)])

        ci.wait()
        cj.wait()
        plsc.subcore_barrier()

        def _issue(b, s):
            pltpu.async_copy(q_hbm.at[dstv.at[b]], qr.at[s], gsem.at[s])
            pltpu.async_copy(kv_hbm.at[srcv.at[b]], kvr.at[s], gsem.at[s])
            pltpu.async_copy(e_hbm.at[pl.ds(tile_base + b * B, B)],
                             er.at[s], gsem.at[s])

        for p in range(GSLOTS):
            _issue(p, p)

        @pl.loop(0, nblk)
        def _blk(b):
            s = lax.rem(b, GSLOTS)
            so = lax.rem(b, OSLOTS)

            pltpu.make_async_copy(q_hbm.at[dstv.at[b]], qr.at[s],
                                  gsem.at[s]).wait()
            pltpu.make_async_copy(kv_hbm.at[srcv.at[b]], kvr.at[s],
                                  gsem.at[s]).wait()
            pltpu.make_async_copy(e_hbm.at[pl.ds(tile_base + b * B, B)],
                                  er.at[s], gsem.at[s]).wait()

            @pl.when(b >= OSLOTS)
            def _():
                pltpu.make_async_copy(outr.at[so], acc.at[dstv.at[b]],
                                      ssem.at[so]).wait()

            @plsc.parallel_loop(0, B, unroll=2)
            def _edge(i):
                den = zeros
                evs = []
                for jp in range(wq // 32 + (1 if wq % 32 else 0)):
                    x = er[s, i, pl.ds(LANES * jp, LANES)]
                    evs.append(plsc.bitcast(jax.lax.shift_left(x, 16), f32))
                    if 32 * jp + 16 < wq:
                        evs.append(plsc.bitcast(x & jnp.int32(-65536), f32))
                for jh in range(nj):
                    sl = pl.ds(LANES * jh, LANES)
                    ev = evs[jh]
                    kvec = kvr[s, i, sl] + ev
                    vvec = kvr[s, i, pl.ds(wq + LANES * jh, LANES)] + ev
                    p = qr[s, i, sl] * kvec
                    pre = plsc.cumsum(p)
                    if not packed:
                        a = jnp.full((LANES,), pre[LANES - 1], f32)
                        ex = jnp.exp(a - SHIFT)
                        outr[so, i, sl] = ex * vvec
                        den = jnp.where(iota == jh, ex, den)
                    else:
                        alo = jnp.full((LANES,), pre[7], f32)
                        ahi = jnp.full((LANES,), pre[LANES - 1], f32) - alo
                        av = jnp.where(iota < 8, alo, ahi)
                        ex = jnp.exp(av - SHIFT)
                        outr[so, i, sl] = ex * vvec
                        exlo = jnp.full((LANES,), ex[0], f32)
                        den = jnp.where(iota == 2 * jh, exlo, den)
                        if 2 * jh + 1 < HEADS:
                            exhi = jnp.full((LANES,), ex[8], f32)
                            den = jnp.where(iota == 2 * jh + 1, exhi, den)
                outr[so, i, pl.ds(wq, LANES)] = den

            pltpu.make_async_copy(outr.at[so], acc.at[dstv.at[b]],
                                  ssem.at[so]).start(add=True)

            @pl.when(b + GSLOTS < nblk)
            def _():
                _issue(b + GSLOTS, s)

        for so in range(OSLOTS):
            pltpu.make_async_copy(outr.at[so], acc.at[dstv.at[0]],
                                  ssem.at[so]).wait()

        plsc.subcore_barrier()

        for t in range(-(-nzc // NS)):
            ck = sid + NS * t

            @pl.when(ck < nzc)
            def _():
                pltpu.sync_copy(acc.at[pl.ds(ck * B, B)],
                                out_hbm.at[cid, pl.ds(ck * B, B)])

    return body(edge_index.reshape(2, E // B, B), qtab, kvtab, etab)


def _tc2_body(acc_ref, skip_ref, wq_ref, bq_ref, wk_ref, bk_ref, wv_ref,
              bv_ref, wsk_ref, bsk_ref, qs_ref, kv_ref, skip2_ref):
    a = acc_ref[0] + acc_ref[1]
    r = a.shape[0]
    num = a[:, :80].reshape(r, HEADS, 16)
    den = a[:, 80:80 + HEADS]
    agg = num / (den[:, :, None] + 1e-30)
    h1 = jnp.maximum(jnp.mean(agg, axis=1) + skip_ref[...], 0.0)
    zpad = jnp.zeros((r, 8), f32)
    q = (jnp.dot(h1, wq_ref[...], preferred_element_type=f32) + bq_ref[...])
    qs_ref[...] = jnp.concatenate([q * (1.0 / jnp.sqrt(8.0)), zpad], axis=1)
    k = jnp.dot(h1, wk_ref[...], preferred_element_type=f32) + bk_ref[...]
    v = jnp.dot(h1, wv_ref[...], preferred_element_type=f32) + bv_ref[...]
    kv_ref[...] = jnp.concatenate([k, zpad, v, zpad], axis=1)
    skip2_ref[...] = jnp.dot(h1, wsk_ref[...], preferred_element_type=f32) + bsk_ref[...]


def _tc2(acc1, skip1, Wq, bq, Wk, bk, Wv, bv, Wsk, bsk):
    R = 2000
    grid = (N // R,)
    full = lambda a: pl.BlockSpec(a.shape, lambda i: (0,) * a.ndim)
    return pl.pallas_call(
        _tc2_body,
        grid=grid,
        in_specs=[pl.BlockSpec((NC, R, 96), lambda i: (0, i, 0)),
                  pl.BlockSpec((R, HID), lambda i: (i, 0)),
                  full(Wq), full(bq), full(Wk), full(bk), full(Wv), full(bv),
                  full(Wsk), full(bsk)],
        out_specs=[pl.BlockSpec((R, 48), lambda i: (i, 0)),
                   pl.BlockSpec((R, 96), lambda i: (i, 0)),
                   pl.BlockSpec((R, 8), lambda i: (i, 0))],
        out_shape=[jax.ShapeDtypeStruct((N, 48), f32),
                   jax.ShapeDtypeStruct((N, 96), f32),
                   jax.ShapeDtypeStruct((N, 8), f32)],
    )(acc1, skip1, Wq, bq, Wk, bk, Wv, bv, Wsk, bsk)


def _tc3_body(acc_ref, skip_ref, w3_ref, b3_ref, w4_ref, b4_ref, out_ref):
    a = acc_ref[0] + acc_ref[1]
    r = a.shape[0]
    num = a[:, :48].reshape(r, 6, 8)[:, :HEADS, :]
    den = a[:, 48:48 + HEADS]
    agg = num / (den[:, :, None] + 1e-30)
    h2 = jnp.maximum(jnp.mean(agg, axis=1) + skip_ref[...], 0.0)
    h3 = jnp.maximum(jnp.dot(h2, w3_ref[...], preferred_element_type=f32) + b3_ref[...], 0.0)
    out_ref[...] = jnp.dot(h3, w4_ref[...], preferred_element_type=f32) + b4_ref[...]


def _tc3(acc2, skip2, W3, b3, W4, b4):
    R = 2000
    grid = (N // R,)
    full = lambda a: pl.BlockSpec(a.shape, lambda i: (0,) * a.ndim)
    return pl.pallas_call(
        _tc3_body,
        grid=grid,
        in_specs=[pl.BlockSpec((NC, R, 64), lambda i: (0, i, 0)),
                  pl.BlockSpec((R, 8), lambda i: (i, 0)),
                  full(W3), full(b3), full(W4), full(b4)],
        out_specs=pl.BlockSpec((R, N_CLASSES), lambda i: (i, 0)),
        out_shape=jax.ShapeDtypeStruct((N, N_CLASSES), f32),
    )(acc2, skip2, W3, b3, W4, b4)


def kernel(x, edge_index, edge_attr,
           Wq1, bq1, Wk1, bk1, Wv1, bv1, We1, Wskip1, bskip1,
           Wq2, bq2, Wk2, bk2, Wv2, bv2, We2, Wskip2, bskip2,
           W3, b3, W4, b4):
    qs1, kv1, skip1 = _tc1a(x, Wq1, bq1, Wk1, bk1, Wv1, bv1, Wskip1, bskip1)
    eaT = edge_attr.T
    z16 = jnp.zeros((D_EDGE, 16), f32)
    w1lo = jnp.concatenate([We1[:, 0:16], We1[:, 32:48], We1[:, 64:80]], axis=1)
    w1hi = jnp.concatenate([We1[:, 16:32], We1[:, 48:64], z16], axis=1)
    w2lo = jnp.concatenate([We2[:, 0:16], We2[:, 32:40], z16[:, :8]], axis=1)
    w2hi = jnp.concatenate([We2[:, 16:32], z16], axis=1)
    e1 = _tc1b(eaT, w1lo, w1hi)
    e2 = _tc1b(eaT, w2lo, w2hi)

    acc1 = _sc_edge_pass(edge_index, qs1, kv1, e1, 80, 96, packed=False, B=B1)
    qs2, kv2, skip2 = _tc2(acc1, skip1, Wq2, bq2, Wk2, bk2, Wv2, bv2,
                           Wskip2, bskip2)
    acc2 = _sc_edge_pass(edge_index, qs2, kv2, e2, 48, 64, packed=True, B=B2)
    return _tc3(acc2, skip2, W3, b3, W4, b4)
